# baseline (speedup 1.0000x reference)
import functools

import jax
import jax.numpy as jnp
from jax import lax
from jax.experimental import pallas as pl
from jax.experimental.pallas import tpu as pltpu
from jax.experimental.pallas import tpu_sc as plsc

N_CODES = 8192
DIM = 32
N_TOK = 16384

TB = 1024
CB = 2048
NT = N_TOK // TB
NCHUNK = N_CODES // CB

_NC, _NS = 2, 16
_NW = _NC * _NS
BPW = N_TOK // _NW
KCH = 128
NJ = BPW // KCH


def _argmin_body(z_ref, w_ref, out_ref):
    zt = z_ref[...].T
    nsq = jnp.sum(zt * zt, axis=0, keepdims=True)
    n = jnp.sqrt(nsq)
    zn = zt / jnp.maximum(n, 1e-12)
    znsq = jnp.sum(zn * zn, axis=0, keepdims=True)

    zn16 = zn.astype(jnp.bfloat16)

    def chunk_min(c):
        wc = w_ref[pl.ds(c * CB, CB), :]
        wsq = jnp.sum(wc * wc, axis=1, keepdims=True)
        mm = lax.dot_general(wc.astype(jnp.bfloat16), zn16,
                             (((1,), (0,)), ((), ())),
                             preferred_element_type=jnp.float32)
        d = (wsq + znsq) - 2.0 * mm
        m = jnp.min(d, axis=0, keepdims=True)
        io = lax.broadcasted_iota(jnp.int32, (CB, TB), 0)
        li = jnp.min(jnp.where(d == m, io, jnp.int32(2**30)),
                     axis=0, keepdims=True) + (c * CB)
        return m, li

    best_v = jnp.full((1, TB), jnp.inf, jnp.float32)
    best_i = jnp.zeros((1, TB), jnp.int32)
    for blk in range(NCHUNK // 2):
        m0, li0 = chunk_min(2 * blk)
        m1, li1 = chunk_min(2 * blk + 1)
        take = m1 < m0
        m = jnp.where(take, m1, m0)
        li = jnp.where(take, li1, li0)
        upd = m < best_v
        best_i = jnp.where(upd, li, best_i)
        best_v = jnp.where(upd, m.astype(jnp.bfloat16).astype(jnp.float32),
                           best_v)
    out_ref[...] = best_i.reshape(1, 1, TB)


def _argmin_indices(z_flat, weight):
    return pl.pallas_call(
        _argmin_body,
        grid=(NT,),
        in_specs=[
            pl.BlockSpec((TB, DIM), lambda i: (i, 0)),
            pl.BlockSpec((N_CODES, DIM), lambda i: (0, 0)),
        ],
        out_specs=pl.BlockSpec((1, 1, TB), lambda i: (i, 0, 0)),
        out_shape=jax.ShapeDtypeStruct((NT, 1, TB), jnp.int32),
    )(z_flat, weight)


@functools.lru_cache(maxsize=None)
def _build_gather():
    mesh = plsc.VectorSubcoreMesh(core_axis_name="c", subcore_axis_name="s")

    @functools.partial(
        pl.kernel,
        mesh=mesh,
        out_type=jax.ShapeDtypeStruct((N_TOK, DIM), jnp.float32),
        scratch_types=[
            pltpu.VMEM((NJ, KCH), jnp.int32),
            pltpu.VMEM((BPW, DIM), jnp.float32),
            pltpu.SemaphoreType.DMA,
        ],
        compiler_params=pltpu.CompilerParams(use_tc_tiling_on_sc=False),
    )
    def _gather_rows(w_hbm, idx_hbm, out_hbm, idx_v, rows_v, sem):
        wid = lax.axis_index("s") * _NC + lax.axis_index("c")
        pltpu.sync_copy(idx_hbm.at[pl.ds(wid * NJ, NJ)], idx_v)
        copies = [
            pltpu.async_copy(w_hbm.at[idx_v.at[j]],
                             rows_v.at[pl.ds(j * KCH, KCH)], sem)
            for j in range(NJ)
        ]
        for c in copies:
            c.wait()
        pltpu.sync_copy(rows_v, out_hbm.at[pl.ds(wid * BPW, BPW)])

    return _gather_rows


def kernel(z, weight):
    idx3 = _argmin_indices(z.reshape(-1, DIM), weight)
    idx_flat = idx3.reshape(N_TOK)
    z_q = _build_gather()(weight, idx_flat.reshape(N_TOK // KCH, KCH))
    return z_q.reshape(z.shape), idx_flat.reshape(z.shape[:-1])


# speedup vs baseline: 1.1284x; 1.1284x over previous
import functools

import jax
import jax.numpy as jnp
from jax import lax
from jax.experimental import pallas as pl
from jax.experimental.pallas import tpu as pltpu
from jax.experimental.pallas import tpu_sc as plsc

N_CODES = 8192
DIM = 32
N_TOK = 16384

TB = 1024
CB = 2048
NT = N_TOK // TB
NCHUNK = N_CODES // CB

_NC, _NS = 2, 16
_NW = _NC * _NS
BPW = N_TOK // _NW
KCH = 128
NJ = BPW // KCH


def _argmin_body(zt_ref, w_ref, out_ref):
    zt = zt_ref[...]
    nsq = jnp.sum(zt * zt, axis=0, keepdims=True)
    n = jnp.sqrt(nsq)
    zn = zt / jnp.maximum(n, 1e-12)
    znsq = jnp.sum(zn * zn, axis=0, keepdims=True)

    zn16 = zn.astype(jnp.bfloat16)

    def chunk_min(c):
        wc = w_ref[pl.ds(c * CB, CB), :]
        wsq = jnp.sum(wc * wc, axis=1, keepdims=True)
        mm = lax.dot_general(wc.astype(jnp.bfloat16), zn16,
                             (((1,), (0,)), ((), ())),
                             preferred_element_type=jnp.float32)
        d = (wsq + znsq) - 2.0 * mm
        m = jnp.min(d, axis=0, keepdims=True)
        io = lax.broadcasted_iota(jnp.int32, (CB, TB), 0)
        li = jnp.min(jnp.where(d == m, io, jnp.int32(2**30)),
                     axis=0, keepdims=True) + (c * CB)
        return m, li

    best_v = jnp.full((1, TB), jnp.inf, jnp.float32)
    best_i = jnp.zeros((1, TB), jnp.int32)
    for blk in range(NCHUNK // 2):
        m0, li0 = chunk_min(2 * blk)
        m1, li1 = chunk_min(2 * blk + 1)
        take = m1 < m0
        m = jnp.where(take, m1, m0)
        li = jnp.where(take, li1, li0)
        upd = m < best_v
        best_i = jnp.where(upd, li, best_i)
        best_v = jnp.where(upd, m.astype(jnp.bfloat16).astype(jnp.float32),
                           best_v)
    out_ref[...] = best_i.reshape(1, 1, TB)


def _argmin_indices(zt, weight):
    return pl.pallas_call(
        _argmin_body,
        grid=(NT,),
        in_specs=[
            pl.BlockSpec((DIM, TB), lambda i: (0, i)),
            pl.BlockSpec((N_CODES, DIM), lambda i: (0, 0)),
        ],
        out_specs=pl.BlockSpec((1, 1, TB), lambda i: (i, 0, 0)),
        out_shape=jax.ShapeDtypeStruct((NT, 1, TB), jnp.int32),
    )(zt, weight)


@functools.lru_cache(maxsize=None)
def _build_gather():
    mesh = plsc.VectorSubcoreMesh(core_axis_name="c", subcore_axis_name="s")

    @functools.partial(
        pl.kernel,
        mesh=mesh,
        out_type=jax.ShapeDtypeStruct((N_TOK, DIM), jnp.float32),
        scratch_types=[
            pltpu.VMEM((NJ, KCH), jnp.int32),
            pltpu.VMEM((BPW, DIM), jnp.float32),
            pltpu.SemaphoreType.DMA,
        ],
        compiler_params=pltpu.CompilerParams(use_tc_tiling_on_sc=False),
    )
    def _gather_rows(w_hbm, idx_hbm, out_hbm, idx_v, rows_v, sem):
        wid = lax.axis_index("s") * _NC + lax.axis_index("c")
        pltpu.sync_copy(idx_hbm.at[pl.ds(wid * NJ, NJ)], idx_v)
        copies = [
            pltpu.async_copy(w_hbm.at[idx_v.at[j]],
                             rows_v.at[pl.ds(j * KCH, KCH)], sem)
            for j in range(NJ)
        ]
        for c in copies:
            c.wait()
        pltpu.sync_copy(rows_v, out_hbm.at[pl.ds(wid * BPW, BPW)])

    return _gather_rows


def kernel(z, weight):
    zt = z.reshape(-1, DIM).T
    idx3 = _argmin_indices(zt, weight)
    idx_flat = idx3.reshape(N_TOK)
    z_q = _build_gather()(weight, idx_flat.reshape(N_TOK // KCH, KCH))
    return z_q.reshape(z.shape), idx_flat.reshape(z.shape[:-1])


# speedup vs baseline: 1.1690x; 1.0360x over previous
import functools

import jax
import jax.numpy as jnp
from jax import lax
from jax.experimental import pallas as pl
from jax.experimental.pallas import tpu as pltpu
from jax.experimental.pallas import tpu_sc as plsc

N_CODES = 8192
DIM = 32
N_TOK = 16384

TB = 1024
CB = 2048
NT = N_TOK // TB
NCHUNK = N_CODES // CB

_NC, _NS = 2, 16
_NW = _NC * _NS
BPW = N_TOK // _NW
KCH = 128
NJ = BPW // KCH


def _argmin_body(zt_ref, w_ref, out_ref):
    zt = zt_ref[...]
    nsq = jnp.sum(zt * zt, axis=0, keepdims=True)
    n = jnp.sqrt(nsq)
    zn = zt / jnp.maximum(n, 1e-12)
    znsq = jnp.sum(zn * zn, axis=0, keepdims=True)

    zn16 = zn.astype(jnp.bfloat16)
    io = lax.broadcasted_iota(jnp.int32, (CB, TB), 0)

    def chunk_min(c):
        wc = w_ref[pl.ds(c * CB, CB), :]
        wsq = jnp.sum(wc * wc, axis=1, keepdims=True)
        mm2 = lax.dot_general((-2.0 * wc).astype(jnp.bfloat16), zn16,
                              (((1,), (0,)), ((), ())),
                              preferred_element_type=jnp.float32)
        d = (wsq + znsq) + mm2
        m = jnp.min(d, axis=0, keepdims=True)
        li = jnp.min(jnp.where(d == m, io, jnp.int32(2**30)),
                     axis=0, keepdims=True) + (c * CB)
        return m, li

    best_v = jnp.full((1, TB), jnp.inf, jnp.float32)
    best_i = jnp.zeros((1, TB), jnp.int32)
    for blk in range(NCHUNK // 2):
        m0, li0 = chunk_min(2 * blk)
        m1, li1 = chunk_min(2 * blk + 1)
        take = m1 < m0
        m = jnp.where(take, m1, m0)
        li = jnp.where(take, li1, li0)
        upd = m < best_v
        best_i = jnp.where(upd, li, best_i)
        best_v = jnp.where(upd, m.astype(jnp.bfloat16).astype(jnp.float32),
                           best_v)
    out_ref[...] = best_i.reshape(1, 1, TB)


def _argmin_indices(zt, weight):
    return pl.pallas_call(
        _argmin_body,
        grid=(NT,),
        in_specs=[
            pl.BlockSpec((DIM, TB), lambda i: (0, i)),
            pl.BlockSpec((N_CODES, DIM), lambda i: (0, 0)),
        ],
        out_specs=pl.BlockSpec((1, 1, TB), lambda i: (i, 0, 0)),
        out_shape=jax.ShapeDtypeStruct((NT, 1, TB), jnp.int32),
    )(zt, weight)


@functools.lru_cache(maxsize=None)
def _build_gather():
    mesh = plsc.VectorSubcoreMesh(core_axis_name="c", subcore_axis_name="s")

    @functools.partial(
        pl.kernel,
        mesh=mesh,
        out_type=jax.ShapeDtypeStruct((N_TOK, DIM), jnp.float32),
        scratch_types=[
            pltpu.VMEM((NJ, KCH), jnp.int32),
            pltpu.VMEM((BPW, DIM), jnp.float32),
            pltpu.SemaphoreType.DMA,
        ],
        compiler_params=pltpu.CompilerParams(use_tc_tiling_on_sc=False),
    )
    def _gather_rows(w_hbm, idx_hbm, out_hbm, idx_v, rows_v, sem):
        wid = lax.axis_index("s") * _NC + lax.axis_index("c")
        pltpu.sync_copy(idx_hbm.at[pl.ds(wid * NJ, NJ)], idx_v)
        copies = [
            pltpu.async_copy(w_hbm.at[idx_v.at[j]],
                             rows_v.at[pl.ds(j * KCH, KCH)], sem)
            for j in range(NJ)
        ]
        for c in copies:
            c.wait()
        pltpu.sync_copy(rows_v, out_hbm.at[pl.ds(wid * BPW, BPW)])

    return _gather_rows


def kernel(z, weight):
    zt = z.reshape(-1, DIM).T
    idx3 = _argmin_indices(zt, weight)
    idx_flat = idx3.reshape(N_TOK)
    z_q = _build_gather()(weight, idx_flat.reshape(N_TOK // KCH, KCH))
    return z_q.reshape(z.shape), idx_flat.reshape(z.shape[:-1])
